# baseline (speedup 1.0000x reference)
import functools

import jax
import jax.numpy as jnp
from jax import lax
from jax.experimental import pallas as pl
from jax.experimental.pallas import tpu as pltpu
from jax.experimental.pallas import tpu_sc as plsc

H = 64
EPS = 1e-12
BBLK = 256
SUB = 128
UNROLL = 4


def _rsqrt_nr(v):
    i = lax.bitcast_convert_type(v, jnp.int32)
    i = jnp.int32(0x5F3759DF) - (i >> 1)
    y = lax.bitcast_convert_type(i, jnp.float32)
    for _ in range(2):
        y = y * (1.5 - 0.5 * v * y * y)
    return y


@functools.cache
def _make_sc_kernel(n_b, n_s):
    info = plsc.get_sparse_core_info()
    nw = info.num_cores * info.num_subcores
    blocks_per_s = n_b // BBLK
    n_units = n_s * blocks_per_s
    per_w = n_units // nw
    assert per_w * nw == n_units and per_w % 2 == 0
    n_pairs = per_w // 2
    ids_per_w = per_w * BBLK

    mesh = plsc.VectorSubcoreMesh(core_axis_name="c", subcore_axis_name="s")

    @functools.partial(
        pl.kernel,
        mesh=mesh,
        compiler_params=pltpu.CompilerParams(
            needs_layout_passes=False, use_tc_tiling_on_sc=True),
        out_type=jax.ShapeDtypeStruct((n_s, H, n_b), jnp.float32),
        scratch_types=[
            pltpu.VMEM((ids_per_w,), jnp.int32),
            pltpu.VMEM((BBLK, 2 * H), jnp.float32),
            pltpu.VMEM((BBLK, 2 * H), jnp.float32),
            pltpu.VMEM((H, BBLK), jnp.float32),
            pltpu.VMEM((H, BBLK), jnp.float32),
            pltpu.VMEM((H,), jnp.float32),
            pltpu.VMEM((H,), jnp.float32),
            pltpu.SemaphoreType.DMA,
            pltpu.SemaphoreType.DMA,
            pltpu.SemaphoreType.DMA,
            pltpu.SemaphoreType.DMA,
        ],
    )
    def k(ids_hbm, table_hbm, gamma_hbm, beta_hbm, out_hbm,
          ids_all, rows0, rows1, st0, st1, gam_v, bet_v,
          gsem0, gsem1, osem0, osem1):
        wid = lax.axis_index("s") * info.num_cores + lax.axis_index("c")
        pltpu.sync_copy(gamma_hbm, gam_v)
        pltpu.sync_copy(beta_hbm, bet_v)
        pltpu.sync_copy(ids_hbm.at[pl.ds(wid * ids_per_w, ids_per_w)],
                        ids_all)
        u_base = wid * per_w
        lane = lax.iota(jnp.int32, 16)
        gs = [gam_v[pl.ds(k16 * 16, 16)] for k16 in range(H // 16)]
        bs = [bet_v[pl.ds(k16 * 16, 16)] for k16 in range(H // 16)]
        ihs = [lane + jnp.int32(k16 * 16) for k16 in range(H // 16)]

        def fire(t, rows_v, gsem):
            base = t * BBLK
            for j in range(BBLK // SUB):
                pltpu.make_async_copy(
                    table_hbm.at[ids_all.at[pl.ds(base + j * SUB, SUB)]],
                    rows_v.at[pl.ds(j * SUB, SUB)], gsem).start()

        def drain(t, rows_v, gsem):
            base = t * BBLK
            for j in range(BBLK // SUB):
                pltpu.make_async_copy(
                    table_hbm.at[ids_all.at[pl.ds(base + j * SUB, SUB)]],
                    rows_v.at[pl.ds(j * SUB, SUB)], gsem).wait()

        def out_ref(t):
            u = u_base + t
            s = u // blocks_per_s
            b0 = (u % blocks_per_s) * BBLK
            return out_hbm.at[s, :, pl.ds(b0, BBLK)]

        def compute(rows_v, st):
            def grp(g, _):
                base = g * UNROLL
                for u in range(UNROLL):
                    r = base + u
                    xs = [rows_v[r, pl.ds(k16 * 16, 16)]
                          for k16 in range(H // 16)]
                    s = jnp.sum(xs[0] + xs[1] + xs[2] + xs[3])
                    q = jnp.sum(xs[0] * xs[0] + xs[1] * xs[1]
                                + xs[2] * xs[2] + xs[3] * xs[3])
                    s_vec = jnp.full((16,), s, jnp.float32)
                    q_vec = jnp.full((16,), q, jnp.float32)
                    mean = s_vec * (1.0 / H)
                    var = q_vec * (1.0 / H) - mean * mean
                    a = _rsqrt_nr(var + EPS)
                    rv = jnp.full((16,), r, jnp.int32)
                    for k16 in range(H // 16):
                        y = (xs[k16] - mean) * a
                        plsc.store_scatter(st, [ihs[k16], rv],
                                           y * gs[k16] + bs[k16])
                return 0

            lax.fori_loop(0, BBLK // UNROLL, grp, 0)

        fire(0, rows0, gsem0)
        fire(1, rows1, gsem1)

        def pair_body(t, _):
            ta = 2 * t
            tb = ta + 1
            drain(ta, rows0, gsem0)

            @pl.when(t > 0)
            def _():
                pltpu.make_async_copy(st0, out_ref(ta - 2), osem0).wait()

            compute(rows0, st0)
            pltpu.make_async_copy(st0, out_ref(ta), osem0).start()

            @pl.when(t < n_pairs - 1)
            def _():
                fire(ta + 2, rows0, gsem0)

            drain(tb, rows1, gsem1)

            @pl.when(t > 0)
            def _():
                pltpu.make_async_copy(st1, out_ref(tb - 2), osem1).wait()

            compute(rows1, st1)
            pltpu.make_async_copy(st1, out_ref(tb), osem1).start()

            @pl.when(t < n_pairs - 1)
            def _():
                fire(tb + 2, rows1, gsem1)

            return 0

        lax.fori_loop(0, n_pairs, pair_body, 0)
        pltpu.make_async_copy(st0, out_ref(per_w - 2), osem0).wait()
        pltpu.make_async_copy(st1, out_ref(per_w - 1), osem1).wait()

    return k


def kernel(input_ids, item_table, ln_gamma, ln_beta):
    b, s = input_ids.shape
    v, h = item_table.shape
    ids_flat = input_ids.T.astype(jnp.int32).reshape(b * s)
    table2 = jnp.pad(item_table, ((0, 0), (0, h)))
    out = _make_sc_kernel(b, s)(ids_flat, table2, ln_gamma, ln_beta)
    return out.transpose(2, 0, 1)


# speedup vs baseline: 1.6810x; 1.6810x over previous
import functools

import jax
import jax.numpy as jnp
from jax import lax
from jax.experimental import pallas as pl
from jax.experimental.pallas import tpu as pltpu
from jax.experimental.pallas import tpu_sc as plsc

H = 64
EPS = 1e-12
CHUNK = 256
SUB = 128
NSUB = CHUNK // SUB
UNROLL = 4


def _rsqrt_nr(v):
    i = lax.bitcast_convert_type(v, jnp.int32)
    i = jnp.int32(0x5F3759DF) - (i >> 1)
    y = lax.bitcast_convert_type(i, jnp.float32)
    for _ in range(2):
        y = y * (1.5 - 0.5 * v * y * y)
    return y


@functools.cache
def _make_sc_kernel(n_rows):
    info = plsc.get_sparse_core_info()
    nw = info.num_cores * info.num_subcores
    per_w = n_rows // nw
    n_chunks = per_w // CHUNK
    assert per_w * nw == n_rows and n_chunks * CHUNK == per_w
    assert n_chunks % 2 == 0
    n_pairs = n_chunks // 2

    mesh = plsc.VectorSubcoreMesh(core_axis_name="c", subcore_axis_name="s")

    @functools.partial(
        pl.kernel,
        mesh=mesh,
        compiler_params=pltpu.CompilerParams(
            needs_layout_passes=False, use_tc_tiling_on_sc=False),
        out_type=jax.ShapeDtypeStruct((n_rows, H), jnp.float32),
        scratch_types=[
            pltpu.VMEM((CHUNK,), jnp.int32),
            pltpu.VMEM((CHUNK,), jnp.int32),
            pltpu.VMEM((CHUNK, H), jnp.float32),
            pltpu.VMEM((CHUNK, H), jnp.float32),
            pltpu.VMEM((CHUNK, H), jnp.float32),
            pltpu.VMEM((CHUNK, H), jnp.float32),
            pltpu.VMEM((H,), jnp.float32),
            pltpu.VMEM((H,), jnp.float32),
            pltpu.SemaphoreType.DMA,
            pltpu.SemaphoreType.DMA,
            pltpu.SemaphoreType.DMA,
            pltpu.SemaphoreType.DMA,
        ],
    )
    def k(ids_hbm, table_hbm, gamma_hbm, beta_hbm, out_hbm,
          idx0, idx1, rows0, rows1, ob0, ob1, gam_v, bet_v,
          gsem0, gsem1, osem0, osem1):
        wid = lax.axis_index("s") * info.num_cores + lax.axis_index("c")
        pltpu.sync_copy(gamma_hbm, gam_v)
        pltpu.sync_copy(beta_hbm, bet_v)
        w_base = wid * per_w
        gs = [gam_v[pl.ds(k16 * 16, 16)] for k16 in range(H // 16)]
        bs = [bet_v[pl.ds(k16 * 16, 16)] for k16 in range(H // 16)]

        def fire(c, idx_v, rows_v, gsem):
            pltpu.sync_copy(ids_hbm.at[pl.ds(w_base + c * CHUNK, CHUNK)],
                            idx_v)
            for j in range(NSUB):
                pltpu.make_async_copy(
                    table_hbm.at[idx_v.at[pl.ds(j * SUB, SUB)]],
                    rows_v.at[pl.ds(j * SUB, SUB)], gsem).start()

        def drain(idx_v, rows_v, gsem):
            for j in range(NSUB):
                pltpu.make_async_copy(
                    table_hbm.at[idx_v.at[pl.ds(j * SUB, SUB)]],
                    rows_v.at[pl.ds(j * SUB, SUB)], gsem).wait()

        def out_start(c, ob, osem):
            pltpu.make_async_copy(
                ob, out_hbm.at[pl.ds(w_base + c * CHUNK, CHUNK)],
                osem).start()

        def out_wait(c, ob, osem):
            pltpu.make_async_copy(
                ob, out_hbm.at[pl.ds(w_base + c * CHUNK, CHUNK)],
                osem).wait()

        def compute(rows_v, ob):
            def quad(i, _):
                r0 = i * UNROLL
                for u in range(UNROLL):
                    r = r0 + u
                    xs = [rows_v[r, pl.ds(k16 * 16, 16)]
                          for k16 in range(H // 16)]
                    s = jnp.sum(xs[0] + xs[1] + xs[2] + xs[3])
                    q = jnp.sum(xs[0] * xs[0] + xs[1] * xs[1]
                                + xs[2] * xs[2] + xs[3] * xs[3])
                    s_vec = jnp.full((16,), s, jnp.float32)
                    q_vec = jnp.full((16,), q, jnp.float32)
                    mean = s_vec * (1.0 / H)
                    var = q_vec * (1.0 / H) - mean * mean
                    a = _rsqrt_nr(var + EPS)
                    for k16 in range(H // 16):
                        y = (xs[k16] - mean) * a
                        ob[r, pl.ds(k16 * 16, 16)] = y * gs[k16] + bs[k16]
                return 0

            lax.fori_loop(0, CHUNK // UNROLL, quad, 0)

        fire(0, idx0, rows0, gsem0)

        def pair_body(t, _):
            ca = 2 * t
            cb = ca + 1
            fire(cb, idx1, rows1, gsem1)
            drain(idx0, rows0, gsem0)

            @pl.when(t > 0)
            def _():
                out_wait(ca - 2, ob0, osem0)

            compute(rows0, ob0)
            out_start(ca, ob0, osem0)

            @pl.when(t < n_pairs - 1)
            def _():
                fire(ca + 2, idx0, rows0, gsem0)

            drain(idx1, rows1, gsem1)

            @pl.when(t > 0)
            def _():
                out_wait(cb - 2, ob1, osem1)

            compute(rows1, ob1)
            out_start(cb, ob1, osem1)
            return 0

        lax.fori_loop(0, n_pairs, pair_body, 0)
        out_wait(n_chunks - 2, ob0, osem0)
        out_wait(n_chunks - 1, ob1, osem1)

    return k


def kernel(input_ids, item_table, ln_gamma, ln_beta):
    b, s = input_ids.shape
    n_rows = b * s
    ids = input_ids.reshape(n_rows).astype(jnp.int32)
    out = _make_sc_kernel(n_rows)(ids, item_table, ln_gamma, ln_beta)
    return out.reshape(b, s, H)
